# baseline (speedup 1.0000x reference)
import functools

import jax
import jax.numpy as jnp
import numpy as np
from jax import lax
from jax.experimental import pallas as pl
from jax.experimental.pallas import tpu as pltpu
from jax.experimental.pallas import tpu_sc as plsc

N = 10000
D_IN = 128
D_H = 64
N_CLS = 16
NUM_LAYERS = 8
ALPHA = 0.1
LAMBDA = 0.5

NC = 2
NS = 16
NW = NC * NS
CH = 128
NCH_TOT = 2500
NCH_BASE = 78
NXTRA = NCH_TOT - NW * NCH_BASE
N_PAD = 10240
RPT = N_PAD // NS
NP2 = N_PAD // 2
PPT = RPT // 2
NSLOT = 8
NRING = 80

_MESH = plsc.VectorSubcoreMesh(core_axis_name="c", subcore_axis_name="s")
_SC_PARAMS = pltpu.CompilerParams(use_tc_tiling_on_sc=False,
                                  needs_layout_passes=False)


def _stage_indices(edge_hbm, wid, src_v, dst_v):
    cbase = wid * NCH_BASE + jnp.minimum(wid, NXTRA)
    extra = wid < NXTRA
    pltpu.sync_copy(edge_hbm.at[0, pl.ds(cbase, NCH_BASE)],
                    src_v.at[pl.ds(0, NCH_BASE)])
    pltpu.sync_copy(edge_hbm.at[1, pl.ds(cbase, NCH_BASE)],
                    dst_v.at[pl.ds(0, NCH_BASE)])

    @pl.when(extra)
    def _():
        pltpu.sync_copy(edge_hbm.at[0, cbase + NCH_BASE], src_v.at[NCH_BASE])
        pltpu.sync_copy(edge_hbm.at[1, cbase + NCH_BASE], dst_v.at[NCH_BASE])
    return extra


@functools.partial(
    pl.kernel,
    out_type=(
        jax.ShapeDtypeStruct((NC, NP2, CH), jnp.float32),
        jax.ShapeDtypeStruct((NC, NP2, CH), jnp.float32),
    ),
    mesh=_MESH,
    scratch_types=[
        pltpu.VMEM((NCH_BASE + 1, CH), jnp.int32),
        pltpu.VMEM((NCH_BASE + 1, CH), jnp.int32),
        pltpu.VMEM((CH,), jnp.float32),
        pltpu.VMEM((RPT,), jnp.float32),
        pltpu.VMEM((RPT,), jnp.float32),
        pltpu.VMEM((PPT, CH), jnp.float32),
        pltpu.VMEM_SHARED((N_PAD,), jnp.float32),
        pltpu.VMEM_SHARED((N_PAD,), jnp.float32),
    ],
    compiler_params=_SC_PARAMS,
)
def _sc_degrees(edge_hbm, dego_hbm, degi_hbm,
                src_v, dst_v, ones_v, do_v, di_v, exp_v, dego_sh, degi_sh):
    c = lax.axis_index("c")
    s = lax.axis_index("s")
    wid = c * NS + s
    extra = _stage_indices(edge_hbm, wid, src_v, dst_v)
    for j in range(CH // 16):
        ones_v[pl.ds(j * 16, 16)] = jnp.ones((16,), jnp.float32)

    def _zero(i, carry):
        do_v[pl.ds(i * 16, 16)] = jnp.zeros((16,), jnp.float32)
        return carry

    lax.fori_loop(0, RPT // 16, _zero, 0)
    pltpu.sync_copy(do_v, dego_sh.at[pl.ds(s * RPT, RPT)])
    pltpu.sync_copy(do_v, degi_sh.at[pl.ds(s * RPT, RPT)])
    plsc.subcore_barrier()

    def _body(ci, carry):
        pltpu.sync_copy(ones_v, dego_sh.at[src_v.at[ci]], add=True)
        pltpu.sync_copy(ones_v, degi_sh.at[dst_v.at[ci]], add=True)
        return carry

    lax.fori_loop(0, NCH_BASE + extra.astype(jnp.int32), _body, 0)
    plsc.subcore_barrier()

    pltpu.sync_copy(dego_sh.at[pl.ds(s * RPT, RPT)], do_v)
    pltpu.sync_copy(degi_sh.at[pl.ds(s * RPT, RPT)], di_v)

    def _expand(deg_v, out_hbm):
        def _egrp(gidx, carry):
            base = gidx * 16
            for k in range(16):
                idx = jnp.full((16,), base + k, jnp.int32)
                vec = plsc.load_gather(deg_v, [idx])
                p = 8 * gidx + k // 2
                for q in range(4):
                    exp_v[p, pl.ds((k % 2) * 64 + q * 16, 16)] = vec
            return carry

        lax.fori_loop(0, RPT // 16, _egrp, 0)
        pltpu.sync_copy(exp_v, out_hbm.at[c, pl.ds(s * PPT, PPT)])

    _expand(do_v, dego_hbm)
    _expand(di_v, degi_hbm)


@functools.partial(
    pl.kernel,
    out_type=jax.ShapeDtypeStruct((NC, N_PAD, D_H), jnp.float32),
    mesh=_MESH,
    scratch_types=[
        pltpu.VMEM((NRING, CH), jnp.int32),
        pltpu.VMEM((NRING, CH), jnp.int32),
        pltpu.VMEM((NSLOT, CH, D_H), jnp.float32),
        pltpu.VMEM_SHARED((N_PAD, D_H), jnp.float32),
        [pltpu.SemaphoreType.DMA] * NSLOT,
        [pltpu.SemaphoreType.DMA] * NSLOT,
    ],
    compiler_params=_SC_PARAMS,
)
def _sc_gather_scatter(g_hbm, edge_hbm, z_hbm, out_hbm,
                       src_v, dst_v, buf_v, agg_sh, gsem, ssem):
    c = lax.axis_index("c")
    s = lax.axis_index("s")
    wid = c * NS + s
    extra = _stage_indices(edge_hbm, wid, src_v, dst_v)

    iota16 = lax.iota(jnp.int32, 16)

    def _fill_dummy(r):
        for j in range(CH // 16):
            src_v[r, pl.ds(j * 16, 16)] = wid * 256 + j * 16 + iota16
            dst_v[r, pl.ds(j * 16, 16)] = (
                N + ((wid * 8 + j) % 15) * 16 + iota16)

    _fill_dummy(NRING - 1)

    @pl.when(jnp.logical_not(extra))
    def _():
        _fill_dummy(NCH_BASE)

    pltpu.sync_copy(z_hbm.at[pl.ds(s * RPT, RPT)],
                    agg_sh.at[pl.ds(s * RPT, RPT)])
    plsc.subcore_barrier()

    for pb in range(5):
        pltpu.async_copy(g_hbm.at[src_v.at[pb]], buf_v.at[pb], gsem[pb])

    def _group(gi, carry):
        for b in range(NSLOT):
            ci = gi * NSLOT + b
            nb = (b + 5) % NSLOT

            @pl.when(ci >= 3)
            def _():
                pltpu.make_async_copy(
                    buf_v.at[nb],
                    agg_sh.at[dst_v.at[ci]],
                    ssem[nb]).wait()

            @pl.when(ci + 5 < NRING)
            def _():
                pltpu.async_copy(g_hbm.at[src_v.at[ci + 5]],
                                 buf_v.at[nb], gsem[nb])

            pltpu.make_async_copy(g_hbm.at[src_v.at[ci]],
                                  buf_v.at[b], gsem[b]).wait()
            pltpu.async_copy(buf_v.at[b], agg_sh.at[dst_v.at[ci]],
                             ssem[b], add=True)
        return carry

    lax.fori_loop(0, NRING // NSLOT, _group, 0)
    for ci in range(NRING - 3, NRING):
        b = ci % NSLOT
        pltpu.make_async_copy(buf_v.at[b], agg_sh.at[dst_v.at[0]],
                              ssem[b]).wait()

    plsc.subcore_barrier()
    pltpu.sync_copy(agg_sh.at[pl.ds(s * RPT, RPT)],
                    out_hbm.at[c, pl.ds(s * RPT, RPT)])


def _tc_mm_body(xp_ref, w_ref, b_ref, h_ref):
    h = jnp.dot(xp_ref[...], w_ref[...], preferred_element_type=jnp.float32)
    h = jnp.maximum(h + b_ref[...][None, :], 0.0)
    h_ref[...] = jnp.concatenate(
        [h, jnp.zeros((NP2 - N // 2, CH), jnp.float32)], axis=0)


def _tc_mm(xp, w2, b2):
    return pl.pallas_call(
        _tc_mm_body,
        out_shape=jax.ShapeDtypeStruct((NP2, CH), jnp.float32),
    )(xp, w2, b2)


def _tc_scale_body(h_ref, go_ref, gi_ref, g_ref, dsrc_ref, ddst_ref):
    dego = go_ref[0] + go_ref[1]
    degi = gi_ref[0] + gi_ref[1]
    dsrc = lax.rsqrt(jnp.where(dego > 0, dego, 1.0))
    ddst = lax.rsqrt(jnp.where(degi > 0, degi, 1.0))
    g_ref[...] = h_ref[...] * dsrc
    dsrc_ref[...] = dsrc
    ddst_ref[...] = ddst


def _tc_scale(h, dego_p, degi_p):
    return pl.pallas_call(
        _tc_scale_body,
        out_shape=(
            jax.ShapeDtypeStruct((NP2, CH), jnp.float32),
            jax.ShapeDtypeStruct((NP2, CH), jnp.float32),
            jax.ShapeDtypeStruct((NP2, CH), jnp.float32),
        ),
    )(h, dego_p, degi_p)


def _tc_layer_body(beta, part_ref, h0_ref, dsrc_ref, ddst_ref, w_ref, g_ref):
    agg = (part_ref[0] + part_ref[1]) * ddst_ref[...]
    feat = (1.0 - ALPHA) * agg + ALPHA * h0_ref[...]
    t = jnp.dot(feat, w_ref[...], preferred_element_type=jnp.float32)
    h = jnp.maximum((1.0 - beta) * feat + beta * t, 0.0)
    g_ref[...] = h * dsrc_ref[...]


def _tc_layer(part, h0, dsrc, ddst, w2, beta):
    return pl.pallas_call(
        functools.partial(_tc_layer_body, beta),
        out_shape=jax.ShapeDtypeStruct((NP2, CH), jnp.float32),
    )(part, h0, dsrc, ddst, w2)


def _tc_last_body(beta, part_ref, h0_ref, ddst_ref, w_ref,
                  fc1w_ref, fc1b_ref, out_ref):
    agg = (part_ref[0, :N // 2, :] + part_ref[1, :N // 2, :]) \
        * ddst_ref[:N // 2, :]
    feat = (1.0 - ALPHA) * agg + ALPHA * h0_ref[:N // 2, :]
    t = jnp.dot(feat, w_ref[...], preferred_element_type=jnp.float32)
    h = jnp.maximum((1.0 - beta) * feat + beta * t, 0.0)
    o = jnp.dot(h, fc1w_ref[...], preferred_element_type=jnp.float32)
    out_ref[...] = jnp.maximum(o + fc1b_ref[...][None, :], 0.0)


def _tc_last(part, h0, ddst, w2, fc1_w2, fc1_b2, beta):
    return pl.pallas_call(
        functools.partial(_tc_last_body, beta),
        out_shape=jax.ShapeDtypeStruct((N // 2, 2 * N_CLS), jnp.float32),
    )(part, h0, ddst, w2, fc1_w2, fc1_b2)


def _blockdiag2(w):
    k, m = w.shape
    z = jnp.zeros((k, m), w.dtype)
    return jnp.concatenate(
        [jnp.concatenate([w, z], axis=1), jnp.concatenate([z, w], axis=1)],
        axis=0)


def kernel(x, edge_index, fc0_w, fc0_b, layer_ws, fc1_w, fc1_b):
    edges = edge_index.reshape(2, NCH_TOT, CH)
    zeros2d = jnp.zeros((N_PAD, D_H), jnp.float32)
    xp = x.reshape(N // 2, 2 * D_IN)
    fc0_w2 = _blockdiag2(fc0_w)
    fc0_b2 = jnp.concatenate([fc0_b, fc0_b])
    fc1_w2 = _blockdiag2(fc1_w)
    fc1_b2 = jnp.concatenate([fc1_b, fc1_b])

    dego_p, degi_p = _sc_degrees(edges)
    h0 = _tc_mm(xp, fc0_w2, fc0_b2)
    g, dsrc, ddst = _tc_scale(h0, dego_p, degi_p)
    for i in range(NUM_LAYERS - 2):
        beta = float(np.log(LAMBDA / (i + 1) + 1.0))
        part = _sc_gather_scatter(g.reshape(N_PAD, D_H), edges, zeros2d)
        part = part.reshape(NC, NP2, CH)
        if i < NUM_LAYERS - 3:
            g = _tc_layer(part, h0, dsrc, ddst, _blockdiag2(layer_ws[i]), beta)
        else:
            out = _tc_last(part, h0, ddst, _blockdiag2(layer_ws[i]),
                           fc1_w2, fc1_b2, beta)
    return out.reshape(N, N_CLS)


# speedup vs baseline: 1.0126x; 1.0126x over previous
import functools

import jax
import jax.numpy as jnp
import numpy as np
from jax import lax
from jax.experimental import pallas as pl
from jax.experimental.pallas import tpu as pltpu
from jax.experimental.pallas import tpu_sc as plsc

N = 10000
D_IN = 128
D_H = 64
N_CLS = 16
NUM_LAYERS = 8
ALPHA = 0.1
LAMBDA = 0.5

NC = 2
NS = 16
NW = NC * NS
CH = 128
NCH_TOT = 2500
NCH_BASE = 78
NXTRA = NCH_TOT - NW * NCH_BASE
N_PAD = 10240
RPT = N_PAD // NS
NP2 = N_PAD // 2
PPT = RPT // 2
NSLOT = 6

_MESH = plsc.VectorSubcoreMesh(core_axis_name="c", subcore_axis_name="s")
_SC_PARAMS = pltpu.CompilerParams(use_tc_tiling_on_sc=False,
                                  needs_layout_passes=False)


def _stage_indices(edge_hbm, wid, src_v, dst_v):
    cbase = wid * NCH_BASE + jnp.minimum(wid, NXTRA)
    extra = wid < NXTRA
    pltpu.sync_copy(edge_hbm.at[0, pl.ds(cbase, NCH_BASE)],
                    src_v.at[pl.ds(0, NCH_BASE)])
    pltpu.sync_copy(edge_hbm.at[1, pl.ds(cbase, NCH_BASE)],
                    dst_v.at[pl.ds(0, NCH_BASE)])

    @pl.when(extra)
    def _():
        pltpu.sync_copy(edge_hbm.at[0, cbase + NCH_BASE], src_v.at[NCH_BASE])
        pltpu.sync_copy(edge_hbm.at[1, cbase + NCH_BASE], dst_v.at[NCH_BASE])
    return extra


@functools.partial(
    pl.kernel,
    out_type=(
        jax.ShapeDtypeStruct((NC, NP2, CH), jnp.float32),
        jax.ShapeDtypeStruct((NC, NP2, CH), jnp.float32),
    ),
    mesh=_MESH,
    scratch_types=[
        pltpu.VMEM((NCH_BASE + 1, CH), jnp.int32),
        pltpu.VMEM((NCH_BASE + 1, CH), jnp.int32),
        pltpu.VMEM((CH,), jnp.float32),
        pltpu.VMEM((RPT,), jnp.float32),
        pltpu.VMEM((RPT,), jnp.float32),
        pltpu.VMEM((PPT, CH), jnp.float32),
        pltpu.VMEM_SHARED((N_PAD,), jnp.float32),
        pltpu.VMEM_SHARED((N_PAD,), jnp.float32),
    ],
    compiler_params=_SC_PARAMS,
)
def _sc_degrees(edge_hbm, dego_hbm, degi_hbm,
                src_v, dst_v, ones_v, do_v, di_v, exp_v, dego_sh, degi_sh):
    c = lax.axis_index("c")
    s = lax.axis_index("s")
    wid = c * NS + s
    extra = _stage_indices(edge_hbm, wid, src_v, dst_v)
    for j in range(CH // 16):
        ones_v[pl.ds(j * 16, 16)] = jnp.ones((16,), jnp.float32)

    def _zero(i, carry):
        do_v[pl.ds(i * 16, 16)] = jnp.zeros((16,), jnp.float32)
        return carry

    lax.fori_loop(0, RPT // 16, _zero, 0)
    pltpu.sync_copy(do_v, dego_sh.at[pl.ds(s * RPT, RPT)])
    pltpu.sync_copy(do_v, degi_sh.at[pl.ds(s * RPT, RPT)])
    plsc.subcore_barrier()

    def _body(ci, carry):
        pltpu.sync_copy(ones_v, dego_sh.at[src_v.at[ci]], add=True)
        pltpu.sync_copy(ones_v, degi_sh.at[dst_v.at[ci]], add=True)
        return carry

    lax.fori_loop(0, NCH_BASE + extra.astype(jnp.int32), _body, 0)
    plsc.subcore_barrier()

    pltpu.sync_copy(dego_sh.at[pl.ds(s * RPT, RPT)], do_v)
    pltpu.sync_copy(degi_sh.at[pl.ds(s * RPT, RPT)], di_v)

    def _expand(deg_v, out_hbm):
        def _egrp(gidx, carry):
            base = gidx * 16
            for k in range(16):
                idx = jnp.full((16,), base + k, jnp.int32)
                vec = plsc.load_gather(deg_v, [idx])
                p = 8 * gidx + k // 2
                for q in range(4):
                    exp_v[p, pl.ds((k % 2) * 64 + q * 16, 16)] = vec
            return carry

        lax.fori_loop(0, RPT // 16, _egrp, 0)
        pltpu.sync_copy(exp_v, out_hbm.at[c, pl.ds(s * PPT, PPT)])

    _expand(do_v, dego_hbm)
    _expand(di_v, degi_hbm)


@functools.partial(
    pl.kernel,
    out_type=jax.ShapeDtypeStruct((NC, N_PAD, D_H), jnp.float32),
    mesh=_MESH,
    scratch_types=[
        pltpu.VMEM((NCH_BASE + 1, CH), jnp.int32),
        pltpu.VMEM((NCH_BASE + 1, CH), jnp.int32),
        pltpu.VMEM((NSLOT, CH, D_H), jnp.float32),
        pltpu.VMEM_SHARED((N_PAD, D_H), jnp.float32),
        [pltpu.SemaphoreType.DMA] * NSLOT,
        [pltpu.SemaphoreType.DMA] * NSLOT,
    ],
    compiler_params=_SC_PARAMS,
)
def _sc_gather_scatter(g_hbm, edge_hbm, z_hbm, out_hbm,
                       src_v, dst_v, buf_v, agg_sh, gsem, ssem):
    c = lax.axis_index("c")
    s = lax.axis_index("s")
    wid = c * NS + s
    extra = _stage_indices(edge_hbm, wid, src_v, dst_v)
    pltpu.sync_copy(z_hbm.at[pl.ds(s * RPT, RPT)],
                    agg_sh.at[pl.ds(s * RPT, RPT)])
    plsc.subcore_barrier()

    for pb in range(4):
        pltpu.async_copy(g_hbm.at[src_v.at[pb]], buf_v.at[pb], gsem[pb])

    def _group(gi, carry):
        for b in range(NSLOT):
            ci = gi * NSLOT + b
            nb = (b + 4) % NSLOT

            @pl.when(ci >= 2)
            def _():
                pltpu.make_async_copy(
                    buf_v.at[nb],
                    agg_sh.at[dst_v.at[ci]],
                    ssem[nb]).wait()

            @pl.when(ci + 4 < NCH_BASE)
            def _():
                pltpu.async_copy(g_hbm.at[src_v.at[ci + 4]],
                                 buf_v.at[nb], gsem[nb])

            pltpu.make_async_copy(g_hbm.at[src_v.at[ci]],
                                  buf_v.at[b], gsem[b]).wait()
            pltpu.async_copy(buf_v.at[b], agg_sh.at[dst_v.at[ci]],
                             ssem[b], add=True)
        return carry

    lax.fori_loop(0, NCH_BASE // NSLOT, _group, 0)
    for ci in range(NCH_BASE - 2, NCH_BASE):
        b = ci % NSLOT
        pltpu.make_async_copy(buf_v.at[b], agg_sh.at[dst_v.at[0]],
                              ssem[b]).wait()

    @pl.when(extra)
    def _():
        pltpu.sync_copy(g_hbm.at[src_v.at[NCH_BASE]], buf_v.at[0])
        pltpu.sync_copy(buf_v.at[0], agg_sh.at[dst_v.at[NCH_BASE]], add=True)

    plsc.subcore_barrier()
    pltpu.sync_copy(agg_sh.at[pl.ds(s * RPT, RPT)],
                    out_hbm.at[c, pl.ds(s * RPT, RPT)])


def _tc_mm_body(xp_ref, w_ref, b_ref, h_ref):
    h = jnp.dot(xp_ref[...], w_ref[...], preferred_element_type=jnp.float32)
    h = jnp.maximum(h + b_ref[...][None, :], 0.0)
    h_ref[...] = jnp.concatenate(
        [h, jnp.zeros((NP2 - N // 2, CH), jnp.float32)], axis=0)


def _tc_mm(xp, w2, b2):
    return pl.pallas_call(
        _tc_mm_body,
        out_shape=jax.ShapeDtypeStruct((NP2, CH), jnp.float32),
    )(xp, w2, b2)


def _tc_scale_body(h_ref, go_ref, gi_ref, y0_ref, w_ref, uinv_ref):
    dego = go_ref[0] + go_ref[1]
    degi = gi_ref[0] + gi_ref[1]
    dego = jnp.where(dego > 0, dego, 1.0)
    dsrc = lax.rsqrt(dego)
    ddst = lax.rsqrt(jnp.where(degi > 0, degi, 1.0))
    y0_ref[...] = h_ref[...] * dsrc
    w_ref[...] = dsrc * ddst
    uinv_ref[...] = jnp.sqrt(dego)


def _tc_scale(h, dego_p, degi_p):
    return pl.pallas_call(
        _tc_scale_body,
        out_shape=(
            jax.ShapeDtypeStruct((NP2, CH), jnp.float32),
            jax.ShapeDtypeStruct((NP2, CH), jnp.float32),
            jax.ShapeDtypeStruct((NP2, CH), jnp.float32),
        ),
    )(h, dego_p, degi_p)


def _tc_layer_body(beta, part_ref, y0_ref, w_ref, wm_ref, g_ref):
    fu = (1.0 - ALPHA) * (part_ref[0] + part_ref[1]) * w_ref[...] \
        + ALPHA * y0_ref[...]
    t = jnp.dot(fu, wm_ref[...], preferred_element_type=jnp.float32)
    g_ref[...] = jnp.maximum((1.0 - beta) * fu + beta * t, 0.0)


def _tc_layer(part, y0, w, wm2, beta):
    return pl.pallas_call(
        functools.partial(_tc_layer_body, beta),
        out_shape=jax.ShapeDtypeStruct((NP2, CH), jnp.float32),
    )(part, y0, w, wm2)


def _tc_last_body(beta, part_ref, y0_ref, w_ref, uinv_ref, wm_ref,
                  fc1w_ref, fc1b_ref, out_ref):
    fu = (1.0 - ALPHA) * (part_ref[0, :N // 2, :]
                          + part_ref[1, :N // 2, :]) * w_ref[:N // 2, :] \
        + ALPHA * y0_ref[:N // 2, :]
    t = jnp.dot(fu, wm_ref[...], preferred_element_type=jnp.float32)
    h = jnp.maximum((1.0 - beta) * fu + beta * t, 0.0) * uinv_ref[:N // 2, :]
    o = jnp.dot(h, fc1w_ref[...], preferred_element_type=jnp.float32)
    out_ref[...] = jnp.maximum(o + fc1b_ref[...][None, :], 0.0)


def _tc_last(part, y0, w, uinv, wm2, fc1_w2, fc1_b2, beta):
    return pl.pallas_call(
        functools.partial(_tc_last_body, beta),
        out_shape=jax.ShapeDtypeStruct((N // 2, 2 * N_CLS), jnp.float32),
    )(part, y0, w, uinv, wm2, fc1_w2, fc1_b2)


def _blockdiag2(w):
    k, m = w.shape
    z = jnp.zeros((k, m), w.dtype)
    return jnp.concatenate(
        [jnp.concatenate([w, z], axis=1), jnp.concatenate([z, w], axis=1)],
        axis=0)


def kernel(x, edge_index, fc0_w, fc0_b, layer_ws, fc1_w, fc1_b):
    edges = edge_index.reshape(2, NCH_TOT, CH)
    zeros2d = jnp.zeros((N_PAD, D_H), jnp.float32)
    xp = x.reshape(N // 2, 2 * D_IN)
    fc0_w2 = _blockdiag2(fc0_w)
    fc0_b2 = jnp.concatenate([fc0_b, fc0_b])
    fc1_w2 = _blockdiag2(fc1_w)
    fc1_b2 = jnp.concatenate([fc1_b, fc1_b])

    dego_p, degi_p = _sc_degrees(edges)
    h0 = _tc_mm(xp, fc0_w2, fc0_b2)
    y0, w, uinv = _tc_scale(h0, dego_p, degi_p)
    g = y0
    for i in range(NUM_LAYERS - 2):
        beta = float(np.log(LAMBDA / (i + 1) + 1.0))
        part = _sc_gather_scatter(g.reshape(N_PAD, D_H), edges, zeros2d)
        part = part.reshape(NC, NP2, CH)
        if i < NUM_LAYERS - 3:
            g = _tc_layer(part, y0, w, _blockdiag2(layer_ws[i]), beta)
        else:
            out = _tc_last(part, y0, w, uinv, _blockdiag2(layer_ws[i]),
                           fc1_w2, fc1_b2, beta)
    return out.reshape(N, N_CLS)


# speedup vs baseline: 1.0339x; 1.0210x over previous
import functools

import jax
import jax.numpy as jnp
import numpy as np
from jax import lax
from jax.experimental import pallas as pl
from jax.experimental.pallas import tpu as pltpu
from jax.experimental.pallas import tpu_sc as plsc

N = 10000
D_IN = 128
D_H = 64
N_CLS = 16
NUM_LAYERS = 8
ALPHA = 0.1
LAMBDA = 0.5

NC = 2
NS = 16
NW = NC * NS
CH = 128
NCH_TOT = 2500
NCH_BASE = 78
NXTRA = NCH_TOT - NW * NCH_BASE
N_PAD = 10240
RPT = N_PAD // NS
NP2 = N_PAD // 2
PPT = RPT // 2
NSLOT = 6

_MESH = plsc.VectorSubcoreMesh(core_axis_name="c", subcore_axis_name="s")
_SC_PARAMS = pltpu.CompilerParams(use_tc_tiling_on_sc=False,
                                  needs_layout_passes=False)


def _stage_indices(edge_hbm, wid, src_v, dst_v):
    cbase = wid * NCH_BASE + jnp.minimum(wid, NXTRA)
    extra = wid < NXTRA
    pltpu.sync_copy(edge_hbm.at[0, pl.ds(cbase, NCH_BASE)],
                    src_v.at[pl.ds(0, NCH_BASE)])
    pltpu.sync_copy(edge_hbm.at[1, pl.ds(cbase, NCH_BASE)],
                    dst_v.at[pl.ds(0, NCH_BASE)])

    @pl.when(extra)
    def _():
        pltpu.sync_copy(edge_hbm.at[0, cbase + NCH_BASE], src_v.at[NCH_BASE])
        pltpu.sync_copy(edge_hbm.at[1, cbase + NCH_BASE], dst_v.at[NCH_BASE])
    return extra


@functools.partial(
    pl.kernel,
    out_type=(
        jax.ShapeDtypeStruct((NC, NP2, CH), jnp.float32),
        jax.ShapeDtypeStruct((NC, NP2, CH), jnp.float32),
    ),
    mesh=_MESH,
    scratch_types=[
        pltpu.VMEM((NCH_BASE + 1, CH), jnp.int32),
        pltpu.VMEM((NCH_BASE + 1, CH), jnp.int32),
        pltpu.VMEM((CH,), jnp.float32),
        pltpu.VMEM((RPT,), jnp.float32),
        pltpu.VMEM((RPT,), jnp.float32),
        pltpu.VMEM((PPT, CH), jnp.float32),
        pltpu.VMEM_SHARED((N_PAD,), jnp.float32),
        pltpu.VMEM_SHARED((N_PAD,), jnp.float32),
        pltpu.SemaphoreType.DMA,
        pltpu.SemaphoreType.DMA,
    ],
    compiler_params=_SC_PARAMS,
)
def _sc_degrees(edge_hbm, dego_hbm, degi_hbm,
                src_v, dst_v, ones_v, do_v, di_v, exp_v, dego_sh, degi_sh,
                osem, isem):
    c = lax.axis_index("c")
    s = lax.axis_index("s")
    wid = c * NS + s
    extra = _stage_indices(edge_hbm, wid, src_v, dst_v)
    for j in range(CH // 16):
        ones_v[pl.ds(j * 16, 16)] = jnp.ones((16,), jnp.float32)

    def _zero(i, carry):
        do_v[pl.ds(i * 16, 16)] = jnp.zeros((16,), jnp.float32)
        return carry

    lax.fori_loop(0, RPT // 16, _zero, 0)
    pltpu.sync_copy(do_v, dego_sh.at[pl.ds(s * RPT, RPT)])
    pltpu.sync_copy(do_v, degi_sh.at[pl.ds(s * RPT, RPT)])
    plsc.subcore_barrier()

    nch = NCH_BASE + extra.astype(jnp.int32)

    def _body(gi, carry):
        base = gi * 4
        for j in range(4):
            ci = jnp.minimum(base + j, nch - 1)
            pltpu.async_copy(ones_v, dego_sh.at[src_v.at[ci]], osem, add=True)
            pltpu.async_copy(ones_v, degi_sh.at[dst_v.at[ci]], isem, add=True)
        for j in range(4):
            pltpu.make_async_copy(ones_v, dego_sh.at[src_v.at[0]],
                                  osem).wait()
            pltpu.make_async_copy(ones_v, degi_sh.at[dst_v.at[0]],
                                  isem).wait()
        return carry

    lax.fori_loop(0, nch // 4, _body, 0)

    def _tail(ci, carry):
        pltpu.sync_copy(ones_v, dego_sh.at[src_v.at[ci]], add=True)
        pltpu.sync_copy(ones_v, degi_sh.at[dst_v.at[ci]], add=True)
        return carry

    lax.fori_loop((nch // 4) * 4, nch, _tail, 0)
    plsc.subcore_barrier()

    pltpu.sync_copy(dego_sh.at[pl.ds(s * RPT, RPT)], do_v)
    pltpu.sync_copy(degi_sh.at[pl.ds(s * RPT, RPT)], di_v)

    def _expand(deg_v, out_hbm):
        def _egrp(gidx, carry):
            base = gidx * 16
            for k in range(16):
                idx = jnp.full((16,), base + k, jnp.int32)
                vec = plsc.load_gather(deg_v, [idx])
                p = 8 * gidx + k // 2
                for q in range(4):
                    exp_v[p, pl.ds((k % 2) * 64 + q * 16, 16)] = vec
            return carry

        lax.fori_loop(0, RPT // 16, _egrp, 0)
        pltpu.sync_copy(exp_v, out_hbm.at[c, pl.ds(s * PPT, PPT)])

    _expand(do_v, dego_hbm)
    _expand(di_v, degi_hbm)


@functools.partial(
    pl.kernel,
    out_type=jax.ShapeDtypeStruct((NC, N_PAD, D_H), jnp.float32),
    mesh=_MESH,
    scratch_types=[
        pltpu.VMEM((NCH_BASE + 1, CH), jnp.int32),
        pltpu.VMEM((NCH_BASE + 1, CH), jnp.int32),
        pltpu.VMEM((NSLOT, CH, D_H), jnp.float32),
        pltpu.VMEM_SHARED((N_PAD, D_H), jnp.float32),
        [pltpu.SemaphoreType.DMA] * NSLOT,
        [pltpu.SemaphoreType.DMA] * NSLOT,
    ],
    compiler_params=_SC_PARAMS,
)
def _sc_gather_scatter(g_hbm, edge_hbm, z_hbm, out_hbm,
                       src_v, dst_v, buf_v, agg_sh, gsem, ssem):
    c = lax.axis_index("c")
    s = lax.axis_index("s")
    wid = c * NS + s
    extra = _stage_indices(edge_hbm, wid, src_v, dst_v)
    pltpu.sync_copy(z_hbm.at[pl.ds(s * RPT, RPT)],
                    agg_sh.at[pl.ds(s * RPT, RPT)])
    plsc.subcore_barrier()

    for pb in range(4):
        pltpu.async_copy(g_hbm.at[src_v.at[pb]], buf_v.at[pb], gsem[pb])

    def _group(gi, carry):
        for b in range(NSLOT):
            ci = gi * NSLOT + b
            nb = (b + 4) % NSLOT

            @pl.when(ci >= 2)
            def _():
                pltpu.make_async_copy(
                    buf_v.at[nb],
                    agg_sh.at[dst_v.at[ci]],
                    ssem[nb]).wait()

            @pl.when(ci + 4 < NCH_BASE)
            def _():
                pltpu.async_copy(g_hbm.at[src_v.at[ci + 4]],
                                 buf_v.at[nb], gsem[nb])

            pltpu.make_async_copy(g_hbm.at[src_v.at[ci]],
                                  buf_v.at[b], gsem[b]).wait()
            pltpu.async_copy(buf_v.at[b], agg_sh.at[dst_v.at[ci]],
                             ssem[b], add=True)
        return carry

    lax.fori_loop(0, NCH_BASE // NSLOT, _group, 0)
    for ci in range(NCH_BASE - 2, NCH_BASE):
        b = ci % NSLOT
        pltpu.make_async_copy(buf_v.at[b], agg_sh.at[dst_v.at[0]],
                              ssem[b]).wait()

    @pl.when(extra)
    def _():
        pltpu.sync_copy(g_hbm.at[src_v.at[NCH_BASE]], buf_v.at[0])
        pltpu.sync_copy(buf_v.at[0], agg_sh.at[dst_v.at[NCH_BASE]], add=True)

    plsc.subcore_barrier()
    pltpu.sync_copy(agg_sh.at[pl.ds(s * RPT, RPT)],
                    out_hbm.at[c, pl.ds(s * RPT, RPT)])


def _tc_mm_body(xp_ref, w_ref, b_ref, h_ref):
    h = jnp.dot(xp_ref[...], w_ref[...], preferred_element_type=jnp.float32)
    h = jnp.maximum(h + b_ref[...][None, :], 0.0)
    h_ref[...] = jnp.concatenate(
        [h, jnp.zeros((NP2 - N // 2, CH), jnp.float32)], axis=0)


def _tc_mm(xp, w2, b2):
    return pl.pallas_call(
        _tc_mm_body,
        out_shape=jax.ShapeDtypeStruct((NP2, CH), jnp.float32),
    )(xp, w2, b2)


def _tc_scale_body(h_ref, go_ref, gi_ref, y0_ref, w_ref, uinv_ref):
    dego = go_ref[0] + go_ref[1]
    degi = gi_ref[0] + gi_ref[1]
    dego = jnp.where(dego > 0, dego, 1.0)
    dsrc = lax.rsqrt(dego)
    ddst = lax.rsqrt(jnp.where(degi > 0, degi, 1.0))
    y0_ref[...] = h_ref[...] * dsrc
    w_ref[...] = dsrc * ddst
    uinv_ref[...] = jnp.sqrt(dego)


def _tc_scale(h, dego_p, degi_p):
    return pl.pallas_call(
        _tc_scale_body,
        out_shape=(
            jax.ShapeDtypeStruct((NP2, CH), jnp.float32),
            jax.ShapeDtypeStruct((NP2, CH), jnp.float32),
            jax.ShapeDtypeStruct((NP2, CH), jnp.float32),
        ),
    )(h, dego_p, degi_p)


def _tc_layer_body(beta, part_ref, y0_ref, w_ref, wm_ref, g_ref):
    fu = (1.0 - ALPHA) * (part_ref[0] + part_ref[1]) * w_ref[...] \
        + ALPHA * y0_ref[...]
    t = jnp.dot(fu, wm_ref[...], preferred_element_type=jnp.float32)
    g_ref[...] = jnp.maximum((1.0 - beta) * fu + beta * t, 0.0)


def _tc_layer(part, y0, w, wm2, beta):
    return pl.pallas_call(
        functools.partial(_tc_layer_body, beta),
        out_shape=jax.ShapeDtypeStruct((NP2, CH), jnp.float32),
    )(part, y0, w, wm2)


def _tc_last_body(beta, part_ref, y0_ref, w_ref, uinv_ref, wm_ref,
                  fc1w_ref, fc1b_ref, out_ref):
    fu = (1.0 - ALPHA) * (part_ref[0, :N // 2, :]
                          + part_ref[1, :N // 2, :]) * w_ref[:N // 2, :] \
        + ALPHA * y0_ref[:N // 2, :]
    t = jnp.dot(fu, wm_ref[...], preferred_element_type=jnp.float32)
    h = jnp.maximum((1.0 - beta) * fu + beta * t, 0.0) * uinv_ref[:N // 2, :]
    o = jnp.dot(h, fc1w_ref[...], preferred_element_type=jnp.float32)
    out_ref[...] = jnp.maximum(o + fc1b_ref[...][None, :], 0.0)


def _tc_last(part, y0, w, uinv, wm2, fc1_w2, fc1_b2, beta):
    return pl.pallas_call(
        functools.partial(_tc_last_body, beta),
        out_shape=jax.ShapeDtypeStruct((N // 2, 2 * N_CLS), jnp.float32),
    )(part, y0, w, uinv, wm2, fc1_w2, fc1_b2)


def _blockdiag2(w):
    k, m = w.shape
    z = jnp.zeros((k, m), w.dtype)
    return jnp.concatenate(
        [jnp.concatenate([w, z], axis=1), jnp.concatenate([z, w], axis=1)],
        axis=0)


def kernel(x, edge_index, fc0_w, fc0_b, layer_ws, fc1_w, fc1_b):
    edges = edge_index.reshape(2, NCH_TOT, CH)
    zeros2d = jnp.zeros((N_PAD, D_H), jnp.float32)
    xp = x.reshape(N // 2, 2 * D_IN)
    fc0_w2 = _blockdiag2(fc0_w)
    fc0_b2 = jnp.concatenate([fc0_b, fc0_b])
    fc1_w2 = _blockdiag2(fc1_w)
    fc1_b2 = jnp.concatenate([fc1_b, fc1_b])

    dego_p, degi_p = _sc_degrees(edges)
    h0 = _tc_mm(xp, fc0_w2, fc0_b2)
    y0, w, uinv = _tc_scale(h0, dego_p, degi_p)
    g = y0
    for i in range(NUM_LAYERS - 2):
        beta = float(np.log(LAMBDA / (i + 1) + 1.0))
        part = _sc_gather_scatter(g.reshape(N_PAD, D_H), edges, zeros2d)
        part = part.reshape(NC, NP2, CH)
        if i < NUM_LAYERS - 3:
            g = _tc_layer(part, y0, w, _blockdiag2(layer_ws[i]), beta)
        else:
            out = _tc_last(part, y0, w, uinv, _blockdiag2(layer_ws[i]),
                           fc1_w2, fc1_b2, beta)
    return out.reshape(N, N_CLS)
